# baseline (speedup 1.0000x reference)
import functools

import jax
import jax.numpy as jnp
from jax import lax
from jax.experimental import pallas as pl
from jax.experimental.pallas import tpu as pltpu
from jax.experimental.pallas import tpu_sc as plsc

NC = 2
NS = 16
L = 16
NW = NC * NS
CHUNK = 64


def _sc_partials(o_flat, b_flat, n_rows, n_ch):
    rows_per_worker = n_rows // NW
    n_chunks = rows_per_worker // CHUNK
    cw = CHUNK * n_ch

    mesh = plsc.VectorSubcoreMesh(core_axis_name="c", subcore_axis_name="s")

    @functools.partial(
        pl.kernel,
        mesh=mesh,
        out_type=jax.ShapeDtypeStruct((NC, NS, L), jnp.float32),
        compiler_params=pltpu.CompilerParams(
            use_tc_tiling_on_sc=False, needs_layout_passes=False),
        scratch_types=[
            pltpu.VMEM((cw,), jnp.float32),
            pltpu.VMEM((cw,), jnp.float32),
            pltpu.VMEM((cw,), jnp.float32),
            pltpu.VMEM((cw,), jnp.float32),
            pltpu.VMEM((L,), jnp.float32),
            pltpu.SemaphoreType.DMA,
            pltpu.SemaphoreType.DMA,
            pltpu.SemaphoreType.DMA,
            pltpu.SemaphoreType.DMA,
        ],
    )
    def sc_kernel(o_hbm, b_hbm, out_hbm, o0, o1, b0, b1, stage,
                  so0, so1, sb0, sb1):
        cid = lax.axis_index("c")
        sid = lax.axis_index("s")
        wid = sid * NC + cid
        base_row = wid * rows_per_worker
        obufs = (o0, o1)
        bbufs = (b0, b1)
        osems = (so0, so1)
        bsems = (sb0, sb1)

        def dma_pair(g, par):
            off = (base_row + g * CHUNK) * n_ch
            oc = pltpu.make_async_copy(
                o_hbm.at[pl.ds(off, cw)], obufs[par], osems[par])
            bc = pltpu.make_async_copy(
                b_hbm.at[pl.ds(off, cw)], bbufs[par], bsems[par])
            return oc, bc

        def start(g, par):
            oc, bc = dma_pair(g, par)
            oc.start()
            bc.start()

        def wait(g, par):
            oc, bc = dma_pair(g, par)
            oc.wait()
            bc.wait()

        start(0, 0)
        start(1, 1)

        lanes = lax.iota(jnp.int32, L)
        neg_inf = jnp.full((L,), -jnp.inf, jnp.float32)
        zeros = jnp.zeros((L,), jnp.float32)

        def chunk_compute(par, loss):
            for gr in range(CHUNK // L):
                idx0 = (gr * L + lanes) * n_ch

                def body(c, carry):
                    idx, rb, ro, rm = carry
                    vb = plsc.load_gather(bbufs[par], [idx])
                    vo = plsc.load_gather(obufs[par], [idx])
                    upd = vb > rb
                    return (idx + 1,
                            jnp.where(upd, vb, rb),
                            jnp.where(upd, vo, ro),
                            jnp.maximum(rm, vo))

                _, _, ro, rm = lax.fori_loop(
                    0, n_ch, body, (idx0, neg_inf, zeros, neg_inf),
                    unroll=4)
                loss = loss + (1.0 - ro / rm)
            return loss

        def loop_body(i, loss):
            for par in range(2):
                g = 2 * i + par
                wait(g, par)

                @pl.when(g + 2 < n_chunks)
                def _():
                    start(g + 2, par)

                loss = chunk_compute(par, loss)
            return loss

        loss = lax.fori_loop(0, n_chunks // 2, loop_body, zeros)
        stage[...] = loss
        pltpu.sync_copy(stage, out_hbm.at[cid, sid])

    return sc_kernel(o_flat, b_flat)


def _tc_sum(partials):
    def body(x_ref, o_ref):
        o_ref[0, 0] = jnp.sum(x_ref[...])

    out = pl.pallas_call(
        body,
        out_shape=jax.ShapeDtypeStruct((1, 1), jnp.float32),
        in_specs=[pl.BlockSpec(memory_space=pltpu.VMEM)],
        out_specs=pl.BlockSpec(memory_space=pltpu.SMEM),
    )(partials)
    return out[0, 0]


def kernel(original, boosted):
    n_ch = original.shape[-1]
    o_flat = original.reshape(-1)
    b_flat = boosted.reshape(-1)
    n_rows = o_flat.size // n_ch
    assert n_rows % (NW * CHUNK) == 0
    partials = _sc_partials(o_flat, b_flat, n_rows, n_ch)
    return _tc_sum(partials)
